# baseline (speedup 1.0000x reference)
import jax
import jax.numpy as jnp
from jax.experimental import pallas as pl
from jax.experimental.pallas import tpu as pltpu

_TILE = 1024
_KSPLIT = 8
_ALPHA = 0.2


def _gat_tile_kernel(x_ref, w_ref, a_ref, *rest):
    adj_refs = rest[:_KSPLIT]
    out_ref = rest[_KSPLIT]
    haug_ref = rest[_KSPLIT + 1]
    s_ref = rest[_KSPLIT + 2]
    t_ref = rest[_KSPLIT + 3]
    f = w_ref.shape[1]
    n = x_ref.shape[0]
    i = pl.program_id(0)

    @pl.when(i == 0)
    def _prep():
        a_vec = a_ref[...]
        na_src = -a_vec[:, :f]
        na_dst = -a_vec[:, f:]
        h_all = jnp.dot(x_ref[...], w_ref[...],
                        preferred_element_type=jnp.float32)
        s = jax.lax.dot_general(h_all, na_src, (((1,), (1,)), ((), ())),
                                preferred_element_type=jnp.float32)
        t = jax.lax.dot_general(na_dst, h_all, (((1,), (1,)), ((), ())),
                                preferred_element_type=jnp.float32)
        s_ref[...] = s.astype(jnp.bfloat16)
        t_ref[...] = t.astype(jnp.bfloat16)
        haug_ref[...] = jnp.concatenate(
            [h_all.astype(jnp.bfloat16),
             jnp.ones((n, 1), dtype=jnp.bfloat16)], axis=1)

    s16 = s_ref[pl.ds(i * _TILE, _TILE), :]
    t16 = t_ref[...]

    c = n // _KSPLIT
    alpha16 = jnp.bfloat16(_ALPHA)
    acc = None
    for k in range(_KSPLIT):
        z = s16 + t16[:, k * c:(k + 1) * c]
        e = jnp.exp(jnp.minimum(z, alpha16 * z))
        e = jnp.where(adj_refs[k][...] != 0, e, jnp.bfloat16(0.0))
        hp_k = jnp.dot(e, haug_ref[k * c:(k + 1) * c, :],
                       preferred_element_type=jnp.float32)
        acc = hp_k if acc is None else acc + hp_k
    hp = acc[:, :f] / acc[:, f:f + 1]
    out_ref[...] = jnp.where(hp > 0, hp, jnp.exp(hp) - 1.0)


def kernel(input, adj, W, a):
    n, in_f = input.shape
    out_f = W.shape[1]
    c = n // _KSPLIT
    adj_i8 = adj.astype(jnp.int8)
    grid = (n // _TILE,)
    in_specs = [
        pl.BlockSpec((n, in_f), lambda i: (0, 0)),
        pl.BlockSpec((in_f, out_f), lambda i: (0, 0)),
        pl.BlockSpec((1, 2 * out_f), lambda i: (0, 0)),
    ]
    for k in range(_KSPLIT):
        in_specs.append(pl.BlockSpec((_TILE, c), lambda i, k=k: (i, k)))
    return pl.pallas_call(
        _gat_tile_kernel,
        grid=grid,
        in_specs=in_specs,
        out_specs=pl.BlockSpec((_TILE, out_f), lambda i: (i, 0)),
        out_shape=jax.ShapeDtypeStruct((n, out_f), jnp.float32),
        scratch_shapes=[
            pltpu.VMEM((n, out_f + 1), jnp.bfloat16),
            pltpu.VMEM((n, 1), jnp.bfloat16),
            pltpu.VMEM((1, n), jnp.bfloat16),
        ],
    )(input, W, a, *([adj_i8] * _KSPLIT))


# speedup vs baseline: 1.0244x; 1.0244x over previous
import jax
import jax.numpy as jnp
from jax.experimental import pallas as pl
from jax.experimental.pallas import tpu as pltpu

_TILE = 1024
_KSPLIT = 2
_ALPHA = 0.2


def _gat_tile_kernel(x_ref, w_ref, a_ref, *rest):
    adj_refs = rest[:_KSPLIT]
    out_ref = rest[_KSPLIT]
    haug_ref = rest[_KSPLIT + 1]
    s_ref = rest[_KSPLIT + 2]
    t_ref = rest[_KSPLIT + 3]
    f = w_ref.shape[1]
    n = x_ref.shape[0]
    i = pl.program_id(0)

    @pl.when(i == 0)
    def _prep():
        a_vec = a_ref[...]
        na_src = -a_vec[:, :f]
        na_dst = -a_vec[:, f:]
        h_all = jnp.dot(x_ref[...], w_ref[...],
                        preferred_element_type=jnp.float32)
        s = jax.lax.dot_general(h_all, na_src, (((1,), (1,)), ((), ())),
                                preferred_element_type=jnp.float32)
        t = jax.lax.dot_general(na_dst, h_all, (((1,), (1,)), ((), ())),
                                preferred_element_type=jnp.float32)
        s_ref[...] = s.astype(jnp.bfloat16)
        t_ref[...] = t.astype(jnp.bfloat16)
        haug_ref[...] = jnp.concatenate(
            [h_all.astype(jnp.bfloat16),
             jnp.ones((n, 1), dtype=jnp.bfloat16)], axis=1)

    s16 = s_ref[pl.ds(i * _TILE, _TILE), :]
    t16 = t_ref[...]

    c = n // _KSPLIT
    alpha16 = jnp.bfloat16(_ALPHA)
    acc = None
    for k in range(_KSPLIT):
        z = s16 + t16[:, k * c:(k + 1) * c]
        e = jnp.exp(jnp.minimum(z, alpha16 * z))
        e = jnp.where(adj_refs[k][...] != 0, e, jnp.bfloat16(0.0))
        hp_k = jnp.dot(e, haug_ref[k * c:(k + 1) * c, :],
                       preferred_element_type=jnp.float32)
        acc = hp_k if acc is None else acc + hp_k
    hp = acc[:, :f] / acc[:, f:f + 1]
    out_ref[...] = jnp.where(hp > 0, hp, jnp.exp(hp) - 1.0)


def kernel(input, adj, W, a):
    n, in_f = input.shape
    out_f = W.shape[1]
    c = n // _KSPLIT
    adj_i8 = adj.astype(jnp.int8)
    grid = (n // _TILE,)
    in_specs = [
        pl.BlockSpec((n, in_f), lambda i: (0, 0)),
        pl.BlockSpec((in_f, out_f), lambda i: (0, 0)),
        pl.BlockSpec((1, 2 * out_f), lambda i: (0, 0)),
    ]
    for k in range(_KSPLIT):
        in_specs.append(pl.BlockSpec((_TILE, c), lambda i, k=k: (i, k)))
    return pl.pallas_call(
        _gat_tile_kernel,
        grid=grid,
        in_specs=in_specs,
        out_specs=pl.BlockSpec((_TILE, out_f), lambda i: (i, 0)),
        out_shape=jax.ShapeDtypeStruct((n, out_f), jnp.float32),
        scratch_shapes=[
            pltpu.VMEM((n, out_f + 1), jnp.bfloat16),
            pltpu.VMEM((n, 1), jnp.bfloat16),
            pltpu.VMEM((1, n), jnp.bfloat16),
        ],
    )(input, W, a, *([adj_i8] * _KSPLIT))
